# baseline (speedup 1.0000x reference)
import functools

import jax
import jax.numpy as jnp
from jax import lax
from jax.experimental import pallas as pl
from jax.experimental.pallas import tpu as pltpu
from jax.experimental.pallas import tpu_sc as plsc

N = 4096
BATCH = 256
NC = 2
NS = 16
L = 16
K = 32
GRP = 3
ROWS_PER_TILE = N // NS
NEG_MIN = float(jnp.finfo(jnp.float32).min)


def _broadcast_lane(vv, k):
    return lax.gather(
        vv, jnp.full((L, 1), k, jnp.int32),
        lax.GatherDimensionNumbers(
            offset_dims=(), collapsed_slice_dims=(0,), start_index_map=(0,)),
        (1,), mode=lax.GatherScatterMode.PROMISE_IN_BOUNDS)


@functools.lru_cache(maxsize=None)
def _build_spmm(nch: int):
    mesh = plsc.VectorSubcoreMesh(
        core_axis_name="c", subcore_axis_name="s", num_cores=NC, num_subcores=NS
    )

    @functools.partial(
        pl.kernel,
        out_type=jax.ShapeDtypeStruct((NC, 2, N, BATCH // 2), jnp.float32),
        mesh=mesh,
        scratch_types=[
            [pltpu.VMEM_SHARED((N, BATCH // 2), jnp.float32)
             for _ in range(2)],
            [pltpu.VMEM((3, K), jnp.int32) for _ in range(3)],
            [pltpu.VMEM((K,), jnp.int32) for _ in range(3)],
            [pltpu.VMEM((K,), jnp.int32) for _ in range(3)],
            [pltpu.VMEM((K, BATCH // 2), jnp.int32) for _ in range(3)],
            [[pltpu.VMEM((K, BATCH // 2), jnp.float32) for _ in range(3)]
             for _ in range(2)],
            [pltpu.SemaphoreType.DMA for _ in range(3)],
            [pltpu.SemaphoreType.DMA for _ in range(3)],
            [pltpu.SemaphoreType.DMA for _ in range(3)],
        ],
    )
    def spmm(xt_hbm, idx_hbm, out_hbm,
             acc_sp, ivbuf, col_v, row_v, gbuf, sbuf, si, sg, ss):
        HB = BATCH // 2
        c = lax.axis_index("c")
        s = lax.axis_index("s")
        r0 = s * ROWS_PER_TILE

        def issue_idx(t, b3):
            pltpu.async_copy(idx_hbm.at[c, s, t], ivbuf[b3], si[b3])

        def wait_idx(t, b3):
            pltpu.make_async_copy(idx_hbm.at[c, s, t], ivbuf[b3],
                                  si[b3]).wait()

        zero = jnp.zeros((L,), jnp.float32)

        def zrow(i, carry):
            for j in range(HB // L):
                sbuf[0][0][i, pl.ds(j * L, L)] = zero
            return carry

        lax.fori_loop(0, 32, zrow, 0)
        for half in range(2):
            for h in range(ROWS_PER_TILE // 32):
                pltpu.sync_copy(sbuf[0][0].at[pl.ds(0, 32)],
                                acc_sp[half].at[pl.ds(r0 + h * 32, 32)])
        plsc.subcore_barrier()

        def step(t, db):
            b = db
            bn3 = (db + 1) % 3
            bn2 = (db + 2) % 3

            pltpu.make_async_copy(xt_hbm.at[col_v[b]], gbuf[b], sg[b]).wait()

            @pl.when(t + 1 < nch)
            def _():
                @pl.when(t >= 2)
                def _():
                    for half in range(2):
                        pltpu.make_async_copy(
                            sbuf[half][bn3],
                            acc_sp[half].at[row_v[bn3]],
                            ss[bn3]).wait()

            @pl.when(t + 2 < nch)
            def _():
                wait_idx(t + 2, bn2)
                for j in range(K // L):
                    col_v[bn2][pl.ds(j * L, L)] = ivbuf[bn2][1,
                                                             pl.ds(j * L, L)]
                pltpu.async_copy(xt_hbm.at[col_v[bn2]], gbuf[bn2], sg[bn2])

            def srow(g, carry):
                base = g * L
                vv = lax.bitcast_convert_type(
                    ivbuf[b][2, pl.ds(base, L)], jnp.float32)
                for k in range(L):
                    v = _broadcast_lane(vv, k)
                    for j in range(BATCH // (2 * L)):
                        half, jj = divmod(j, HB // (2 * L))
                        pw = gbuf[b][base + k, pl.ds(j * L, L)]
                        a0 = lax.bitcast_convert_type(
                            lax.shift_left(pw, 16), jnp.float32)
                        a1 = lax.bitcast_convert_type(
                            pw & jnp.int32(-65536), jnp.float32)
                        sbuf[half][b][base + k, pl.ds(jj * 2 * L, L)] = a0 * v
                        sbuf[half][b][base + k,
                                      pl.ds(jj * 2 * L + L, L)] = a1 * v
                return carry

            lax.fori_loop(0, K // L, srow, 0)
            for j in range(K // L):
                row_v[b][pl.ds(j * L, L)] = ivbuf[b][0, pl.ds(j * L, L)]
            for half in range(2):
                pltpu.async_copy(sbuf[half][b], acc_sp[half].at[row_v[b]],
                                 ss[b], add=True)

            @pl.when(t + 3 < nch)
            def _():
                issue_idx(t + 3, b)

        issue_idx(0, 0)
        issue_idx(1, 1)
        issue_idx(2, 2)
        for tt in range(2):
            wait_idx(tt, tt)
            for j in range(K // L):
                col_v[tt][pl.ds(j * L, L)] = ivbuf[tt][1, pl.ds(j * L, L)]
            pltpu.async_copy(xt_hbm.at[col_v[tt]], gbuf[tt], sg[tt])

        def group(g, carry):
            for db in range(GRP):
                step(g * GRP + db, db)
            return carry

        lax.fori_loop(0, nch // GRP, group, 0)

        for tl in (nch - 3, nch - 2, nch - 1):
            for half in range(2):
                pltpu.make_async_copy(sbuf[half][tl % 3],
                                      acc_sp[half].at[row_v[tl % 3]],
                                      ss[tl % 3]).wait()
        plsc.subcore_barrier()

        for half in range(2):
            pltpu.sync_copy(acc_sp[half].at[pl.ds(r0, ROWS_PER_TILE)],
                            out_hbm.at[c, half, pl.ds(r0, ROWS_PER_TILE)])

    return spmm


def _merge_body(p_ref, o_ref):
    for half in range(2):
        a = p_ref[0, half] + p_ref[1, half]
        a = jnp.where(jnp.isinf(a), jnp.float32(NEG_MIN), a)
        o_ref[pl.ds(half * (BATCH // 2), BATCH // 2), :] = a.T


@functools.lru_cache(maxsize=None)
def _build_merge(rb: int):
    return pl.pallas_call(
        _merge_body,
        grid=(N // rb,),
        in_specs=[pl.BlockSpec((NC, 2, rb, BATCH // 2),
                               lambda i: (0, 0, i, 0))],
        out_specs=pl.BlockSpec((BATCH, rb), lambda i: (0, i)),
        out_shape=jax.ShapeDtypeStruct((BATCH, N), jnp.float32),
    )


def kernel(x, rows, cols, vals):
    nnz = rows.shape[0]
    unit = NC * NS * K * GRP
    e_pad = -(-nnz // unit) * unit
    pad = e_pad - nnz
    if pad:
        fill = (jnp.arange(pad, dtype=jnp.int32) * 17) % N
        rows_p = jnp.concatenate([rows, fill])
        cols_p = jnp.concatenate([cols, fill])
        vals_p = jnp.concatenate([vals, jnp.zeros((pad,), jnp.float32)])
    else:
        rows_p, cols_p, vals_p = rows, cols, vals
    per_tile = e_pad // (NC * NS)
    nch = per_tile // K

    xt = (x.T.astype(jnp.bfloat16)
             .reshape(N, BATCH // 32, 2, 16)
             .transpose(0, 1, 3, 2)
             .reshape(N, BATCH // 2, 2))
    xt = lax.bitcast_convert_type(xt, jnp.int32)

    idx3 = (jnp.stack([rows_p, cols_p,
                       lax.bitcast_convert_type(vals_p, jnp.int32)])
               .reshape(3, NC, NS, nch, K)
               .transpose(1, 2, 3, 0, 4))
    parts = _build_spmm(nch)(xt, idx3)
    return _build_merge(512)(parts)


# speedup vs baseline: 1.5971x; 1.5971x over previous
import functools

import jax
import jax.numpy as jnp
from jax import lax
from jax.experimental import pallas as pl
from jax.experimental.pallas import tpu as pltpu
from jax.experimental.pallas import tpu_sc as plsc

N = 4096
BATCH = 256
NC = 2
NS = 16
L = 16
K = 32
GRP = 3
ROWS_PER_TILE = N // NS
NEG_MIN = float(jnp.finfo(jnp.float32).min)


def _broadcast_lane(vv, k):
    return lax.gather(
        vv, jnp.full((L, 1), k, jnp.int32),
        lax.GatherDimensionNumbers(
            offset_dims=(), collapsed_slice_dims=(0,), start_index_map=(0,)),
        (1,), mode=lax.GatherScatterMode.PROMISE_IN_BOUNDS)


@functools.lru_cache(maxsize=None)
def _build_spmm(nch: int):
    mesh = plsc.VectorSubcoreMesh(
        core_axis_name="c", subcore_axis_name="s", num_cores=NC, num_subcores=NS
    )

    @functools.partial(
        pl.kernel,
        out_type=jax.ShapeDtypeStruct((NC, 2, N, BATCH // 2), jnp.float32),
        mesh=mesh,
        scratch_types=[
            [pltpu.VMEM_SHARED((N, BATCH // 2), jnp.float32)
             for _ in range(2)],
            [pltpu.VMEM((K,), jnp.int32) for _ in range(3)],
            [pltpu.VMEM((K,), jnp.int32) for _ in range(3)],
            [pltpu.VMEM((K,), jnp.float32) for _ in range(3)],
            [pltpu.VMEM((K, BATCH // 2), jnp.int32) for _ in range(3)],
            [[pltpu.VMEM((K, BATCH // 2), jnp.float32) for _ in range(3)]
             for _ in range(2)],
            [pltpu.SemaphoreType.DMA for _ in range(3)],
            [pltpu.SemaphoreType.DMA for _ in range(3)],
            [pltpu.SemaphoreType.DMA for _ in range(3)],
            [pltpu.SemaphoreType.DMA for _ in range(3)],
            [pltpu.SemaphoreType.DMA for _ in range(3)],
        ],
    )
    def spmm(xt_hbm, rows_hbm, cols_hbm, vals_hbm, out_hbm,
             acc_sp, col_v, row_v, val_v, gbuf, sbuf, sic, siv, sir, sg, ss):
        HB = BATCH // 2
        c = lax.axis_index("c")
        s = lax.axis_index("s")
        r0 = s * ROWS_PER_TILE

        def issue_col(t, b3):
            pltpu.async_copy(cols_hbm.at[c, s, t], col_v[b3], sic[b3])

        def wait_col(t, b3):
            pltpu.make_async_copy(cols_hbm.at[c, s, t], col_v[b3],
                                  sic[b3]).wait()

        def issue_val(t, b3):
            pltpu.async_copy(vals_hbm.at[c, s, t], val_v[b3], siv[b3])

        def wait_val(t, b3):
            pltpu.make_async_copy(vals_hbm.at[c, s, t], val_v[b3],
                                  siv[b3]).wait()

        zero = jnp.zeros((L,), jnp.float32)

        def zrow(i, carry):
            for j in range(HB // L):
                sbuf[0][0][i, pl.ds(j * L, L)] = zero
            return carry

        lax.fori_loop(0, 32, zrow, 0)
        for half in range(2):
            for h in range(ROWS_PER_TILE // 32):
                pltpu.sync_copy(sbuf[0][0].at[pl.ds(0, 32)],
                                acc_sp[half].at[pl.ds(r0 + h * 32, 32)])
        plsc.subcore_barrier()

        def step(t, db):
            b = db
            bn3 = (db + 1) % 3
            bn2 = (db + 2) % 3

            pltpu.make_async_copy(xt_hbm.at[col_v[b]], gbuf[b], sg[b]).wait()

            @pl.when(t + 3 < nch)
            def _():
                issue_col(t + 3, b)

            @pl.when(t + 1 < nch)
            def _():
                @pl.when(t >= 2)
                def _():
                    for half in range(2):
                        pltpu.make_async_copy(
                            sbuf[half][bn3],
                            acc_sp[half].at[row_v[bn3]],
                            ss[bn3]).wait()

                pltpu.async_copy(rows_hbm.at[c, s, t + 1], row_v[bn3],
                                 sir[bn3])

            @pl.when(t + 2 < nch)
            def _():
                wait_col(t + 2, bn2)
                pltpu.async_copy(xt_hbm.at[col_v[bn2]], gbuf[bn2], sg[bn2])
                issue_val(t + 2, bn2)

            wait_val(t, b)

            def srow(g, carry):
                base = g * L
                vv = val_v[b][pl.ds(base, L)]
                for k in range(L):
                    v = _broadcast_lane(vv, k)
                    for j in range(BATCH // (2 * L)):
                        half, jj = divmod(j, HB // (2 * L))
                        pw = gbuf[b][base + k, pl.ds(j * L, L)]
                        a0 = lax.bitcast_convert_type(
                            lax.shift_left(pw, 16), jnp.float32)
                        a1 = lax.bitcast_convert_type(
                            pw & jnp.int32(-65536), jnp.float32)
                        sbuf[half][b][base + k, pl.ds(jj * 2 * L, L)] = a0 * v
                        sbuf[half][b][base + k,
                                      pl.ds(jj * 2 * L + L, L)] = a1 * v
                return carry

            lax.fori_loop(0, K // L, srow, 0)
            pltpu.make_async_copy(rows_hbm.at[c, s, t], row_v[b],
                                  sir[b]).wait()
            for half in range(2):
                pltpu.async_copy(sbuf[half][b], acc_sp[half].at[row_v[b]],
                                 ss[b], add=True)

        issue_col(0, 0)
        issue_col(1, 1)
        issue_col(2, 2)
        issue_val(0, 0)
        issue_val(1, 1)
        pltpu.async_copy(rows_hbm.at[c, s, 0], row_v[0], sir[0])
        wait_col(0, 0)
        pltpu.async_copy(xt_hbm.at[col_v[0]], gbuf[0], sg[0])
        wait_col(1, 1)
        pltpu.async_copy(xt_hbm.at[col_v[1]], gbuf[1], sg[1])

        def group(g, carry):
            for db in range(GRP):
                step(g * GRP + db, db)
            return carry

        lax.fori_loop(0, nch // GRP, group, 0)

        for tl in (nch - 3, nch - 2, nch - 1):
            for half in range(2):
                pltpu.make_async_copy(sbuf[half][tl % 3],
                                      acc_sp[half].at[row_v[tl % 3]],
                                      ss[tl % 3]).wait()
        plsc.subcore_barrier()

        for half in range(2):
            pltpu.sync_copy(acc_sp[half].at[pl.ds(r0, ROWS_PER_TILE)],
                            out_hbm.at[c, half, pl.ds(r0, ROWS_PER_TILE)])

    return spmm


def _merge_body(p_ref, o_ref):
    for half in range(2):
        a = p_ref[0, half] + p_ref[1, half]
        a = jnp.where(jnp.isinf(a), jnp.float32(NEG_MIN), a)
        o_ref[pl.ds(half * (BATCH // 2), BATCH // 2), :] = a.T


@functools.lru_cache(maxsize=None)
def _build_merge(rb: int):
    return pl.pallas_call(
        _merge_body,
        grid=(N // rb,),
        in_specs=[pl.BlockSpec((NC, 2, rb, BATCH // 2),
                               lambda i: (0, 0, i, 0))],
        out_specs=pl.BlockSpec((BATCH, rb), lambda i: (0, i)),
        out_shape=jax.ShapeDtypeStruct((BATCH, N), jnp.float32),
    )


def kernel(x, rows, cols, vals):
    nnz = rows.shape[0]
    unit = NC * NS * K * GRP
    e_pad = -(-nnz // unit) * unit
    pad = e_pad - nnz
    if pad:
        fill = (jnp.arange(pad, dtype=jnp.int32) * 17) % N
        rows_p = jnp.concatenate([rows, fill])
        cols_p = jnp.concatenate([cols, fill])
        vals_p = jnp.concatenate([vals, jnp.zeros((pad,), jnp.float32)])
    else:
        rows_p, cols_p, vals_p = rows, cols, vals
    per_tile = e_pad // (NC * NS)
    nch = per_tile // K

    xt = (x.T.astype(jnp.bfloat16)
             .reshape(N, BATCH // 32, 2, 16)
             .transpose(0, 1, 3, 2)
             .reshape(N, BATCH // 2, 2))
    xt = lax.bitcast_convert_type(xt, jnp.int32)

    rows2 = rows_p.reshape(NC, NS, nch, K)
    cols2 = cols_p.reshape(NC, NS, nch, K)
    vals2 = vals_p.reshape(NC, NS, nch, K)
    parts = _build_spmm(nch)(xt, rows2, cols2, vals2)
    return _build_merge(512)(parts)
